# baseline (speedup 1.0000x reference)
import functools

import jax
import jax.numpy as jnp
from jax import lax
from jax.experimental import pallas as pl
from jax.experimental.pallas import tpu as pltpu
from jax.experimental.pallas import tpu_sc as plsc

N = 10000
E = 320000
F = 128

NC = 2
NS = 16
NW = NC * NS
EPW = E // NW
CHUNK = 40
NCHUNK = EPW // CHUNK
NBUF = 5
NGROUP = NCHUNK // NBUF
RPT = 624
TAIL_BASE = RPT * NS
TAIL = N - TAIL_BASE

_SC_PARAMS = pltpu.CompilerParams(use_tc_tiling_on_sc=False)


def _sc_aggregate():
    mesh = plsc.VectorSubcoreMesh(
        core_axis_name="c", subcore_axis_name="s",
        num_cores=NC, num_subcores=NS)

    scratch = [
        pltpu.VMEM((NCHUNK, CHUNK), jnp.int32),
        pltpu.VMEM((NCHUNK, CHUNK), jnp.int32),
        pltpu.VMEM((NBUF * CHUNK, F), jnp.float32),
        pltpu.VMEM_SHARED((N, F), jnp.float32),
    ] + [pltpu.SemaphoreType.DMA] * (2 * NBUF)

    def body(x_hbm, srcg_hbm, dstg_hbm, out_hbm, didx, sidx, rows, accum,
             *sems):
        gsems, ssems = sems[:NBUF], sems[NBUF:]
        c = lax.axis_index("c")
        s = lax.axis_index("s")
        wid = s * NC + c
        row_base = s * RPT

        pltpu.sync_copy(dstg_hbm.at[wid], didx)
        pltpu.sync_copy(srcg_hbm.at[wid], sidx)

        def zrows(r, carry):
            for k in range(F // 16):
                rows[r, pl.ds(k * 16, 16)] = jnp.zeros((16,), jnp.float32)
            return carry
        lax.fori_loop(0, CHUNK, zrows, 0)

        zsrc = rows.at[pl.ds(0, CHUNK)]
        nfull = RPT // CHUNK
        rem = RPT - nfull * CHUNK
        for j in range(nfull):
            pltpu.sync_copy(zsrc, accum.at[pl.ds(row_base + j * CHUNK, CHUNK)])
        pltpu.sync_copy(rows.at[pl.ds(0, rem)],
                        accum.at[pl.ds(row_base + nfull * CHUNK, rem)])

        @pl.when(s == NS - 1)
        def _zero_tail():
            pltpu.sync_copy(rows.at[pl.ds(0, TAIL)],
                            accum.at[pl.ds(TAIL_BASE, TAIL)])

        plsc.subcore_barrier()

        PF = NBUF - 2

        def rslot(b):
            return rows.at[pl.ds(b * CHUNK, CHUNK)]

        for b in range(PF):
            pltpu.async_copy(x_hbm.at[sidx.at[b]], rslot(b), gsems[b])

        def group(g, carry):
            for b in range(NBUF):
                cc = g * NBUF + b
                pltpu.make_async_copy(
                    x_hbm.at[sidx.at[cc]], rslot(b), gsems[b]).wait()
                pltpu.async_copy(
                    rslot(b), accum.at[didx.at[cc]], ssems[b], add=True)

                bp = (b + PF) % NBUF

                @pl.when(cc >= 2)
                def _drain_prev():
                    pltpu.make_async_copy(
                        rslot(bp), accum.at[didx.at[cc - 2]],
                        ssems[bp]).wait()

                @pl.when(cc + PF < NCHUNK)
                def _prefetch():
                    pltpu.async_copy(
                        x_hbm.at[sidx.at[cc + PF]], rslot(bp), gsems[bp])
            return carry
        lax.fori_loop(0, NGROUP, group, 0)

        for cc in (NCHUNK - 2, NCHUNK - 1):
            pltpu.make_async_copy(
                rslot(cc % NBUF), accum.at[didx.at[cc]],
                ssems[cc % NBUF]).wait()

        plsc.subcore_barrier()

        pltpu.sync_copy(accum.at[pl.ds(row_base, RPT)],
                        out_hbm.at[c, pl.ds(row_base, RPT)])

        @pl.when(s == NS - 1)
        def _write_tail():
            pltpu.sync_copy(accum.at[pl.ds(TAIL_BASE, TAIL)],
                            out_hbm.at[c, pl.ds(TAIL_BASE, TAIL)])

    return pl.kernel(body, out_type=jax.ShapeDtypeStruct((NC, N, F),
                                                         jnp.float32),
                     mesh=mesh, scratch_types=scratch,
                     compiler_params=_SC_PARAMS)


DEGW = 16
DCHUNK = 128
DNCH = EPW // DCHUNK
DTAIL = EPW - DNCH * DCHUNK
DNB = 6
DNG = DNCH // DNB


def _sc_degree():
    mesh = plsc.VectorSubcoreMesh(
        core_axis_name="c", subcore_axis_name="s",
        num_cores=NC, num_subcores=NS)

    scratch = [
        pltpu.VMEM((DNCH + 1, DCHUNK), jnp.int32),
        pltpu.VMEM((DCHUNK, DEGW), jnp.float32),
        pltpu.VMEM_SHARED((N, DEGW), jnp.float32),
        pltpu.SemaphoreType.DMA,
    ]

    def body(dstg_hbm, dstt_hbm, out_hbm, didx, ones, accum, ssem):
        c = lax.axis_index("c")
        s = lax.axis_index("s")
        wid = s * NC + c
        row_base = s * RPT

        pltpu.sync_copy(dstg_hbm.at[wid], didx.at[pl.ds(0, DNCH)])
        pltpu.sync_copy(dstt_hbm.at[wid], didx.at[DNCH, pl.ds(0, DTAIL)])

        def zrows(r, carry):
            ones[r, pl.ds(0, DEGW)] = jnp.zeros((DEGW,), jnp.float32)
            return carry
        lax.fori_loop(0, DCHUNK, zrows, 0)

        nfull = RPT // DCHUNK
        rem = RPT - nfull * DCHUNK
        for j in range(nfull):
            pltpu.sync_copy(ones,
                            accum.at[pl.ds(row_base + j * DCHUNK, DCHUNK)])
        pltpu.sync_copy(ones.at[pl.ds(0, rem)],
                        accum.at[pl.ds(row_base + nfull * DCHUNK, rem)])

        @pl.when(s == NS - 1)
        def _zero_tail():
            pltpu.sync_copy(ones.at[pl.ds(0, TAIL)],
                            accum.at[pl.ds(TAIL_BASE, TAIL)])

        def frows(r, carry):
            ones[r, pl.ds(0, DEGW)] = jnp.full((DEGW,), 1.0, jnp.float32)
            return carry
        lax.fori_loop(0, DCHUNK, frows, 0)

        plsc.subcore_barrier()

        def dgroup(g, carry):
            descs = []
            for b in range(DNB):
                cc = g * DNB + b
                descs.append(pltpu.async_copy(
                    ones, accum.at[didx.at[cc]], ssem, add=True))
            for d in descs:
                d.wait()
            return carry
        lax.fori_loop(0, DNG, dgroup, 0)

        pltpu.sync_copy(ones.at[pl.ds(0, DTAIL)],
                        accum.at[didx.at[DNCH, pl.ds(0, DTAIL)]], add=True)

        plsc.subcore_barrier()

        pltpu.sync_copy(accum.at[pl.ds(row_base, RPT)],
                        out_hbm.at[c, pl.ds(row_base, RPT)])

        @pl.when(s == NS - 1)
        def _write_tail():
            pltpu.sync_copy(accum.at[pl.ds(TAIL_BASE, TAIL)],
                            out_hbm.at[c, pl.ds(TAIL_BASE, TAIL)])

    return pl.kernel(body, out_type=jax.ShapeDtypeStruct((NC, N, DEGW),
                                                         jnp.float32),
                     mesh=mesh, scratch_types=scratch,
                     compiler_params=_SC_PARAMS)


BN = 1000


def _tc1_body(sp_ref, dp_ref, x_ref, wl_ref, bl_ref, wr_ref, o_ref):
    summed = sp_ref[0] + sp_ref[1]
    deg = dp_ref[0, :, 0:1] + dp_ref[1, :, 0:1]
    mean = summed * (1.0 / jnp.maximum(deg, 1.0))
    h = (jnp.dot(mean, wl_ref[...], preferred_element_type=jnp.float32)
         + bl_ref[...]
         + jnp.dot(x_ref[...], wr_ref[...], preferred_element_type=jnp.float32))
    o_ref[...] = jnp.maximum(h, 0.0)


def _tc2_body(sp_ref, dp_ref, h_ref, wl_ref, bl_ref, wr_ref, w3_ref, b3_ref,
              o_ref):
    summed = sp_ref[0] + sp_ref[1]
    deg = dp_ref[0, :, 0:1] + dp_ref[1, :, 0:1]
    mean = summed * (1.0 / jnp.maximum(deg, 1.0))
    h2 = (jnp.dot(mean, wl_ref[...], preferred_element_type=jnp.float32)
          + bl_ref[...]
          + jnp.dot(h_ref[...], wr_ref[...], preferred_element_type=jnp.float32))
    o_ref[...] = jnp.sum(h2 * w3_ref[...], axis=1, keepdims=True) + b3_ref[0, 0]


def _tc_layer1(sp, dp, x, wlT, bl, wrT):
    grid = (N // BN,)
    return pl.pallas_call(
        _tc1_body,
        grid=grid,
        in_specs=[
            pl.BlockSpec((NC, BN, F), lambda i: (0, i, 0)),
            pl.BlockSpec((NC, BN, DEGW), lambda i: (0, i, 0)),
            pl.BlockSpec((BN, F), lambda i: (i, 0)),
            pl.BlockSpec((F, F), lambda i: (0, 0)),
            pl.BlockSpec((1, F), lambda i: (0, 0)),
            pl.BlockSpec((F, F), lambda i: (0, 0)),
        ],
        out_specs=pl.BlockSpec((BN, F), lambda i: (i, 0)),
        out_shape=jax.ShapeDtypeStruct((N, F), jnp.float32),
    )(sp, dp, x, wlT, bl, wrT)


def _tc_layer2(sp, dp, h, wlT, bl, wrT, w3, b3):
    grid = (N // BN,)
    return pl.pallas_call(
        _tc2_body,
        grid=grid,
        in_specs=[
            pl.BlockSpec((NC, BN, F), lambda i: (0, i, 0)),
            pl.BlockSpec((NC, BN, DEGW), lambda i: (0, i, 0)),
            pl.BlockSpec((BN, F), lambda i: (i, 0)),
            pl.BlockSpec((F, F), lambda i: (0, 0)),
            pl.BlockSpec((1, F), lambda i: (0, 0)),
            pl.BlockSpec((F, F), lambda i: (0, 0)),
            pl.BlockSpec((1, F), lambda i: (0, 0)),
            pl.BlockSpec((1, 1), lambda i: (0, 0), memory_space=pltpu.SMEM),
        ],
        out_specs=pl.BlockSpec((BN, 1), lambda i: (i, 0)),
        out_shape=jax.ShapeDtypeStruct((N, 1), jnp.float32),
    )(sp, dp, h, wlT, bl, wrT, w3, b3)


@functools.cache
def _sc_kernels():
    return _sc_aggregate(), _sc_degree()


def kernel(x_feature, edge_index, W1l, b1l, W1r, W2l, b2l, W2r, W3, b3):
    agg, deg_count = _sc_kernels()
    srcg = edge_index[0].reshape(NW, NCHUNK, CHUNK)
    dstg = edge_index[1].reshape(NW, NCHUNK, CHUNK)
    dstw = edge_index[1].reshape(NW, EPW)
    dst_main = dstw[:, :DNCH * DCHUNK].reshape(NW, DNCH, DCHUNK)
    dst_tail = dstw[:, DNCH * DCHUNK:]
    dp = deg_count(dst_main, dst_tail)
    sp1 = agg(x_feature, srcg, dstg)
    h = _tc_layer1(sp1, dp, x_feature, W1l.T, b1l[None, :], W1r.T)
    sp2 = agg(h, srcg, dstg)
    return _tc_layer2(sp2, dp, h, W2l.T, b2l[None, :], W2r.T, W3,
                      b3.reshape(1, 1))


# speedup vs baseline: 1.1411x; 1.1411x over previous
import functools

import jax
import jax.numpy as jnp
from jax import lax
from jax.experimental import pallas as pl
from jax.experimental.pallas import tpu as pltpu
from jax.experimental.pallas import tpu_sc as plsc

N = 10000
E = 320000
F = 128

NC = 2
NS = 16
NW = NC * NS
EPW = E // NW
CHUNK = 40
NCHUNK = EPW // CHUNK
NBUF = 5
NGROUP = NCHUNK // NBUF
RPT = 624
TAIL_BASE = RPT * NS
TAIL = N - TAIL_BASE

_SC_PARAMS = pltpu.CompilerParams(use_tc_tiling_on_sc=False)


def _sc_aggregate():
    mesh = plsc.VectorSubcoreMesh(
        core_axis_name="c", subcore_axis_name="s",
        num_cores=NC, num_subcores=NS)

    scratch = [
        pltpu.VMEM((NCHUNK, CHUNK), jnp.int32),
        pltpu.VMEM((NCHUNK, CHUNK), jnp.int32),
        pltpu.VMEM((NBUF * CHUNK, F), jnp.float32),
        pltpu.VMEM_SHARED((N, F), jnp.float32),
        pltpu.SemaphoreType.DMA,
    ] + [pltpu.SemaphoreType.DMA] * NBUF

    def body(x_hbm, srcg_hbm, dstg_hbm, out_hbm, didx, sidx, rows, accum,
             ssem, *gsems):
        c = lax.axis_index("c")
        s = lax.axis_index("s")
        wid = s * NC + c
        row_base = s * RPT

        pltpu.sync_copy(dstg_hbm.at[wid], didx)
        pltpu.sync_copy(srcg_hbm.at[wid], sidx)

        def zrows(r, carry):
            for k in range(F // 16):
                rows[r, pl.ds(k * 16, 16)] = jnp.zeros((16,), jnp.float32)
            return carry
        lax.fori_loop(0, CHUNK, zrows, 0)

        zsrc = rows.at[pl.ds(0, CHUNK)]
        nfull = RPT // CHUNK
        rem = RPT - nfull * CHUNK
        zdescs = [
            pltpu.async_copy(zsrc,
                             accum.at[pl.ds(row_base + j * CHUNK, CHUNK)],
                             ssem)
            for j in range(nfull)]
        zdescs.append(
            pltpu.async_copy(rows.at[pl.ds(0, rem)],
                             accum.at[pl.ds(row_base + nfull * CHUNK, rem)],
                             ssem))
        for d in zdescs:
            d.wait()

        @pl.when(s == NS - 1)
        def _zero_tail():
            pltpu.sync_copy(rows.at[pl.ds(0, TAIL)],
                            accum.at[pl.ds(TAIL_BASE, TAIL)])

        plsc.subcore_barrier()

        def rslot(b):
            return rows.at[pl.ds(b * CHUNK, CHUNK)]

        for b in range(NBUF):
            pltpu.async_copy(x_hbm.at[sidx.at[b]], rslot(b), gsems[b])

        def group(g, carry):
            for b in range(NBUF):
                cc = g * NBUF + b
                pltpu.make_async_copy(
                    x_hbm.at[sidx.at[cc]], rslot(b), gsems[b]).wait()
                sd = pltpu.async_copy(
                    rslot(b), accum.at[didx.at[cc]], ssem, add=True)
                sd.wait()

                @pl.when(g < NGROUP - 1)
                def _prefetch():
                    pltpu.async_copy(
                        x_hbm.at[sidx.at[cc + NBUF]], rslot(b), gsems[b])
            return carry
        lax.fori_loop(0, NGROUP, group, 0)

        plsc.subcore_barrier()

        pltpu.sync_copy(accum.at[pl.ds(row_base, RPT)],
                        out_hbm.at[c, pl.ds(row_base, RPT)])

        @pl.when(s == NS - 1)
        def _write_tail():
            pltpu.sync_copy(accum.at[pl.ds(TAIL_BASE, TAIL)],
                            out_hbm.at[c, pl.ds(TAIL_BASE, TAIL)])

    return pl.kernel(body, out_type=jax.ShapeDtypeStruct((NC, N, F),
                                                         jnp.float32),
                     mesh=mesh, scratch_types=scratch,
                     compiler_params=_SC_PARAMS)


DEGW = 16
DCHUNK = 128
DNCH = EPW // DCHUNK
DTAIL = EPW - DNCH * DCHUNK
DNB = 6
DNG = DNCH // DNB


def _sc_degree():
    mesh = plsc.VectorSubcoreMesh(
        core_axis_name="c", subcore_axis_name="s",
        num_cores=NC, num_subcores=NS)

    scratch = [
        pltpu.VMEM((DNCH + 1, DCHUNK), jnp.int32),
        pltpu.VMEM((DCHUNK, DEGW), jnp.float32),
        pltpu.VMEM_SHARED((N, DEGW), jnp.float32),
        pltpu.SemaphoreType.DMA,
    ]

    def body(dstg_hbm, dstt_hbm, out_hbm, didx, ones, accum, ssem):
        c = lax.axis_index("c")
        s = lax.axis_index("s")
        wid = s * NC + c
        row_base = s * RPT

        pltpu.sync_copy(dstg_hbm.at[wid], didx.at[pl.ds(0, DNCH)])
        pltpu.sync_copy(dstt_hbm.at[wid], didx.at[DNCH, pl.ds(0, DTAIL)])

        def zrows(r, carry):
            ones[r, pl.ds(0, DEGW)] = jnp.zeros((DEGW,), jnp.float32)
            return carry
        lax.fori_loop(0, DCHUNK, zrows, 0)

        nfull = RPT // DCHUNK
        rem = RPT - nfull * DCHUNK
        zdescs = [
            pltpu.async_copy(ones,
                             accum.at[pl.ds(row_base + j * DCHUNK, DCHUNK)],
                             ssem)
            for j in range(nfull)]
        zdescs.append(
            pltpu.async_copy(ones.at[pl.ds(0, rem)],
                             accum.at[pl.ds(row_base + nfull * DCHUNK, rem)],
                             ssem))
        for d in zdescs:
            d.wait()

        @pl.when(s == NS - 1)
        def _zero_tail():
            pltpu.sync_copy(ones.at[pl.ds(0, TAIL)],
                            accum.at[pl.ds(TAIL_BASE, TAIL)])

        def frows(r, carry):
            ones[r, pl.ds(0, DEGW)] = jnp.full((DEGW,), 1.0, jnp.float32)
            return carry
        lax.fori_loop(0, DCHUNK, frows, 0)

        plsc.subcore_barrier()

        def dgroup(g, carry):
            descs = []
            for b in range(DNB):
                cc = g * DNB + b
                descs.append(pltpu.async_copy(
                    ones, accum.at[didx.at[cc]], ssem, add=True))
            for d in descs:
                d.wait()
            return carry
        lax.fori_loop(0, DNG, dgroup, 0)

        pltpu.sync_copy(ones.at[pl.ds(0, DTAIL)],
                        accum.at[didx.at[DNCH, pl.ds(0, DTAIL)]], add=True)

        plsc.subcore_barrier()

        pltpu.sync_copy(accum.at[pl.ds(row_base, RPT)],
                        out_hbm.at[c, pl.ds(row_base, RPT)])

        @pl.when(s == NS - 1)
        def _write_tail():
            pltpu.sync_copy(accum.at[pl.ds(TAIL_BASE, TAIL)],
                            out_hbm.at[c, pl.ds(TAIL_BASE, TAIL)])

    return pl.kernel(body, out_type=jax.ShapeDtypeStruct((NC, N, DEGW),
                                                         jnp.float32),
                     mesh=mesh, scratch_types=scratch,
                     compiler_params=_SC_PARAMS)


BN = 2000


def _tc1_body(sp_ref, dp_ref, x_ref, wl_ref, bl_ref, wr_ref, o_ref):
    summed = sp_ref[0] + sp_ref[1]
    deg = dp_ref[0, :, 0:1] + dp_ref[1, :, 0:1]
    mean = summed * (1.0 / jnp.maximum(deg, 1.0))
    h = (jnp.dot(mean, wl_ref[...], preferred_element_type=jnp.float32)
         + bl_ref[...]
         + jnp.dot(x_ref[...], wr_ref[...], preferred_element_type=jnp.float32))
    o_ref[...] = jnp.maximum(h, 0.0)


def _tc2_body(sp_ref, dp_ref, h_ref, wl_ref, bl_ref, wr_ref, w3_ref, b3_ref,
              o_ref):
    summed = sp_ref[0] + sp_ref[1]
    deg = dp_ref[0, :, 0:1] + dp_ref[1, :, 0:1]
    mean = summed * (1.0 / jnp.maximum(deg, 1.0))
    h2 = (jnp.dot(mean, wl_ref[...], preferred_element_type=jnp.float32)
          + bl_ref[...]
          + jnp.dot(h_ref[...], wr_ref[...], preferred_element_type=jnp.float32))
    o_ref[...] = jnp.sum(h2 * w3_ref[...], axis=1, keepdims=True) + b3_ref[0, 0]


def _tc_layer1(sp, dp, x, wlT, bl, wrT):
    grid = (N // BN,)
    return pl.pallas_call(
        _tc1_body,
        grid=grid,
        in_specs=[
            pl.BlockSpec((NC, BN, F), lambda i: (0, i, 0)),
            pl.BlockSpec((NC, BN, DEGW), lambda i: (0, i, 0)),
            pl.BlockSpec((BN, F), lambda i: (i, 0)),
            pl.BlockSpec((F, F), lambda i: (0, 0)),
            pl.BlockSpec((1, F), lambda i: (0, 0)),
            pl.BlockSpec((F, F), lambda i: (0, 0)),
        ],
        out_specs=pl.BlockSpec((BN, F), lambda i: (i, 0)),
        out_shape=jax.ShapeDtypeStruct((N, F), jnp.float32),
    )(sp, dp, x, wlT, bl, wrT)


def _tc_layer2(sp, dp, h, wlT, bl, wrT, w3, b3):
    grid = (N // BN,)
    return pl.pallas_call(
        _tc2_body,
        grid=grid,
        in_specs=[
            pl.BlockSpec((NC, BN, F), lambda i: (0, i, 0)),
            pl.BlockSpec((NC, BN, DEGW), lambda i: (0, i, 0)),
            pl.BlockSpec((BN, F), lambda i: (i, 0)),
            pl.BlockSpec((F, F), lambda i: (0, 0)),
            pl.BlockSpec((1, F), lambda i: (0, 0)),
            pl.BlockSpec((F, F), lambda i: (0, 0)),
            pl.BlockSpec((1, F), lambda i: (0, 0)),
            pl.BlockSpec((1, 1), lambda i: (0, 0), memory_space=pltpu.SMEM),
        ],
        out_specs=pl.BlockSpec((BN, 1), lambda i: (i, 0)),
        out_shape=jax.ShapeDtypeStruct((N, 1), jnp.float32),
    )(sp, dp, h, wlT, bl, wrT, w3, b3)


@functools.cache
def _sc_kernels():
    return _sc_aggregate(), _sc_degree()


def kernel(x_feature, edge_index, W1l, b1l, W1r, W2l, b2l, W2r, W3, b3):
    agg, deg_count = _sc_kernels()
    srcg = edge_index[0].reshape(NW, NCHUNK, CHUNK)
    dstg = edge_index[1].reshape(NW, NCHUNK, CHUNK)
    dstw = edge_index[1].reshape(NW, EPW)
    dst_main = dstw[:, :DNCH * DCHUNK].reshape(NW, DNCH, DCHUNK)
    dst_tail = dstw[:, DNCH * DCHUNK:]
    dp = deg_count(dst_main, dst_tail)
    sp1 = agg(x_feature, srcg, dstg)
    h = _tc_layer1(sp1, dp, x_feature, W1l.T, b1l[None, :], W1r.T)
    sp2 = agg(h, srcg, dstg)
    return _tc_layer2(sp2, dp, h, W2l.T, b2l[None, :], W2r.T, W3,
                      b3.reshape(1, 1))


# speedup vs baseline: 1.1528x; 1.0103x over previous
import functools

import jax
import jax.numpy as jnp
from jax import lax
from jax.experimental import pallas as pl
from jax.experimental.pallas import tpu as pltpu
from jax.experimental.pallas import tpu_sc as plsc

N = 10000
E = 320000
F = 128

NC = 2
NS = 16
NW = NC * NS
EPW = E // NW
CHUNK = 40
NCHUNK = EPW // CHUNK
NBUF = 5
NGROUP = NCHUNK // NBUF
RPT = 624
TAIL_BASE = RPT * NS
TAIL = N - TAIL_BASE

_SC_PARAMS = pltpu.CompilerParams(use_tc_tiling_on_sc=False)


def _sc_aggregate():
    mesh = plsc.VectorSubcoreMesh(
        core_axis_name="c", subcore_axis_name="s",
        num_cores=NC, num_subcores=NS)

    scratch = [
        pltpu.VMEM((NCHUNK, CHUNK), jnp.int32),
        pltpu.VMEM((NCHUNK, CHUNK), jnp.int32),
        pltpu.VMEM((NBUF * CHUNK, F), jnp.float32),
        pltpu.VMEM_SHARED((N, F), jnp.float32),
        pltpu.SemaphoreType.DMA,
    ] + [pltpu.SemaphoreType.DMA] * NBUF

    def body(x_hbm, srcg_hbm, dstg_hbm, out_hbm, didx, sidx, rows, accum,
             ssem, *gsems):
        c = lax.axis_index("c")
        s = lax.axis_index("s")
        wid = s * NC + c
        row_base = s * RPT

        pd1 = pltpu.async_copy(dstg_hbm.at[wid], didx, ssem)
        pd2 = pltpu.async_copy(srcg_hbm.at[wid], sidx, ssem)

        def zrows(r, carry):
            for k in range(F // 16):
                rows[r, pl.ds(k * 16, 16)] = jnp.zeros((16,), jnp.float32)
            return carry
        lax.fori_loop(0, CHUNK, zrows, 0)
        pd1.wait()
        pd2.wait()

        zsrc = rows.at[pl.ds(0, CHUNK)]
        nfull = RPT // CHUNK
        rem = RPT - nfull * CHUNK
        zdescs = [
            pltpu.async_copy(zsrc,
                             accum.at[pl.ds(row_base + j * CHUNK, CHUNK)],
                             ssem)
            for j in range(nfull)]
        zdescs.append(
            pltpu.async_copy(rows.at[pl.ds(0, rem)],
                             accum.at[pl.ds(row_base + nfull * CHUNK, rem)],
                             ssem))
        for d in zdescs:
            d.wait()

        @pl.when(s == NS - 1)
        def _zero_tail():
            pltpu.sync_copy(rows.at[pl.ds(0, TAIL)],
                            accum.at[pl.ds(TAIL_BASE, TAIL)])

        plsc.subcore_barrier()

        def rslot(b):
            return rows.at[pl.ds(b * CHUNK, CHUNK)]

        for b in range(NBUF):
            pltpu.async_copy(x_hbm.at[sidx.at[b]], rslot(b), gsems[b])

        def group(g, carry):
            for b in range(NBUF):
                cc = g * NBUF + b
                pltpu.make_async_copy(
                    x_hbm.at[sidx.at[cc]], rslot(b), gsems[b]).wait()
                sd = pltpu.async_copy(
                    rslot(b), accum.at[didx.at[cc]], ssem, add=True)
                sd.wait()

                @pl.when(g < NGROUP - 1)
                def _prefetch():
                    pltpu.async_copy(
                        x_hbm.at[sidx.at[cc + NBUF]], rslot(b), gsems[b])
            return carry
        lax.fori_loop(0, NGROUP, group, 0)

        plsc.subcore_barrier()

        pltpu.sync_copy(accum.at[pl.ds(row_base, RPT)],
                        out_hbm.at[c, pl.ds(row_base, RPT)])

        @pl.when(s == NS - 1)
        def _write_tail():
            pltpu.sync_copy(accum.at[pl.ds(TAIL_BASE, TAIL)],
                            out_hbm.at[c, pl.ds(TAIL_BASE, TAIL)])

    return pl.kernel(body, out_type=jax.ShapeDtypeStruct((NC, N, F),
                                                         jnp.float32),
                     mesh=mesh, scratch_types=scratch,
                     compiler_params=_SC_PARAMS)


DEGW = 16
DCHUNK = 128
DNCH = EPW // DCHUNK
DTAIL = EPW - DNCH * DCHUNK
DNB = 6
DNG = DNCH // DNB


def _sc_degree():
    mesh = plsc.VectorSubcoreMesh(
        core_axis_name="c", subcore_axis_name="s",
        num_cores=NC, num_subcores=NS)

    scratch = [
        pltpu.VMEM((DNCH + 1, DCHUNK), jnp.int32),
        pltpu.VMEM((DCHUNK, DEGW), jnp.float32),
        pltpu.VMEM_SHARED((N, DEGW), jnp.float32),
        pltpu.SemaphoreType.DMA,
    ]

    def body(dstg_hbm, dstt_hbm, out_hbm, didx, ones, accum, ssem):
        c = lax.axis_index("c")
        s = lax.axis_index("s")
        wid = s * NC + c
        row_base = s * RPT

        pd1 = pltpu.async_copy(dstg_hbm.at[wid], didx.at[pl.ds(0, DNCH)],
                               ssem)
        pd2 = pltpu.async_copy(dstt_hbm.at[wid],
                               didx.at[DNCH, pl.ds(0, DTAIL)], ssem)

        def zrows(r, carry):
            ones[r, pl.ds(0, DEGW)] = jnp.zeros((DEGW,), jnp.float32)
            return carry
        lax.fori_loop(0, DCHUNK, zrows, 0)
        pd1.wait()
        pd2.wait()

        nfull = RPT // DCHUNK
        rem = RPT - nfull * DCHUNK
        zdescs = [
            pltpu.async_copy(ones,
                             accum.at[pl.ds(row_base + j * DCHUNK, DCHUNK)],
                             ssem)
            for j in range(nfull)]
        zdescs.append(
            pltpu.async_copy(ones.at[pl.ds(0, rem)],
                             accum.at[pl.ds(row_base + nfull * DCHUNK, rem)],
                             ssem))
        for d in zdescs:
            d.wait()

        @pl.when(s == NS - 1)
        def _zero_tail():
            pltpu.sync_copy(ones.at[pl.ds(0, TAIL)],
                            accum.at[pl.ds(TAIL_BASE, TAIL)])

        def frows(r, carry):
            ones[r, pl.ds(0, DEGW)] = jnp.full((DEGW,), 1.0, jnp.float32)
            return carry
        lax.fori_loop(0, DCHUNK, frows, 0)

        plsc.subcore_barrier()

        def dgroup(g, carry):
            descs = []
            for b in range(DNB):
                cc = g * DNB + b
                descs.append(pltpu.async_copy(
                    ones, accum.at[didx.at[cc]], ssem, add=True))
            for d in descs:
                d.wait()
            return carry
        lax.fori_loop(0, DNG, dgroup, 0)

        pltpu.sync_copy(ones.at[pl.ds(0, DTAIL)],
                        accum.at[didx.at[DNCH, pl.ds(0, DTAIL)]], add=True)

        plsc.subcore_barrier()

        pltpu.sync_copy(accum.at[pl.ds(row_base, RPT)],
                        out_hbm.at[c, pl.ds(row_base, RPT)])

        @pl.when(s == NS - 1)
        def _write_tail():
            pltpu.sync_copy(accum.at[pl.ds(TAIL_BASE, TAIL)],
                            out_hbm.at[c, pl.ds(TAIL_BASE, TAIL)])

    return pl.kernel(body, out_type=jax.ShapeDtypeStruct((NC, N, DEGW),
                                                         jnp.float32),
                     mesh=mesh, scratch_types=scratch,
                     compiler_params=_SC_PARAMS)


BN = 2000


def _tc1_body(sp_ref, dp_ref, x_ref, wl_ref, bl_ref, wr_ref, o_ref):
    summed = sp_ref[0] + sp_ref[1]
    deg = dp_ref[0, :, 0:1] + dp_ref[1, :, 0:1]
    mean = summed * (1.0 / jnp.maximum(deg, 1.0))
    h = (jnp.dot(mean, wl_ref[...], preferred_element_type=jnp.float32)
         + bl_ref[...]
         + jnp.dot(x_ref[...], wr_ref[...], preferred_element_type=jnp.float32))
    o_ref[...] = jnp.maximum(h, 0.0)


def _tc2_body(sp_ref, dp_ref, h_ref, wl_ref, bl_ref, wr_ref, w3_ref, b3_ref,
              o_ref):
    summed = sp_ref[0] + sp_ref[1]
    deg = dp_ref[0, :, 0:1] + dp_ref[1, :, 0:1]
    mean = summed * (1.0 / jnp.maximum(deg, 1.0))
    h2 = (jnp.dot(mean, wl_ref[...], preferred_element_type=jnp.float32)
          + bl_ref[...]
          + jnp.dot(h_ref[...], wr_ref[...], preferred_element_type=jnp.float32))
    o_ref[...] = jnp.sum(h2 * w3_ref[...], axis=1, keepdims=True) + b3_ref[0, 0]


def _tc_layer1(sp, dp, x, wlT, bl, wrT):
    grid = (N // BN,)
    return pl.pallas_call(
        _tc1_body,
        grid=grid,
        in_specs=[
            pl.BlockSpec((NC, BN, F), lambda i: (0, i, 0)),
            pl.BlockSpec((NC, BN, DEGW), lambda i: (0, i, 0)),
            pl.BlockSpec((BN, F), lambda i: (i, 0)),
            pl.BlockSpec((F, F), lambda i: (0, 0)),
            pl.BlockSpec((1, F), lambda i: (0, 0)),
            pl.BlockSpec((F, F), lambda i: (0, 0)),
        ],
        out_specs=pl.BlockSpec((BN, F), lambda i: (i, 0)),
        out_shape=jax.ShapeDtypeStruct((N, F), jnp.float32),
    )(sp, dp, x, wlT, bl, wrT)


def _tc_layer2(sp, dp, h, wlT, bl, wrT, w3, b3):
    grid = (N // BN,)
    return pl.pallas_call(
        _tc2_body,
        grid=grid,
        in_specs=[
            pl.BlockSpec((NC, BN, F), lambda i: (0, i, 0)),
            pl.BlockSpec((NC, BN, DEGW), lambda i: (0, i, 0)),
            pl.BlockSpec((BN, F), lambda i: (i, 0)),
            pl.BlockSpec((F, F), lambda i: (0, 0)),
            pl.BlockSpec((1, F), lambda i: (0, 0)),
            pl.BlockSpec((F, F), lambda i: (0, 0)),
            pl.BlockSpec((1, F), lambda i: (0, 0)),
            pl.BlockSpec((1, 1), lambda i: (0, 0), memory_space=pltpu.SMEM),
        ],
        out_specs=pl.BlockSpec((BN, 1), lambda i: (i, 0)),
        out_shape=jax.ShapeDtypeStruct((N, 1), jnp.float32),
    )(sp, dp, h, wlT, bl, wrT, w3, b3)


@functools.cache
def _sc_kernels():
    return _sc_aggregate(), _sc_degree()


def kernel(x_feature, edge_index, W1l, b1l, W1r, W2l, b2l, W2r, W3, b3):
    agg, deg_count = _sc_kernels()
    srcg = edge_index[0].reshape(NW, NCHUNK, CHUNK)
    dstg = edge_index[1].reshape(NW, NCHUNK, CHUNK)
    dstw = edge_index[1].reshape(NW, EPW)
    dst_main = dstw[:, :DNCH * DCHUNK].reshape(NW, DNCH, DCHUNK)
    dst_tail = dstw[:, DNCH * DCHUNK:]
    dp = deg_count(dst_main, dst_tail)
    sp1 = agg(x_feature, srcg, dstg)
    h = _tc_layer1(sp1, dp, x_feature, W1l.T, b1l[None, :], W1r.T)
    sp2 = agg(h, srcg, dstg)
    return _tc_layer2(sp2, dp, h, W2l.T, b2l[None, :], W2r.T, W3,
                      b3.reshape(1, 1))
